# baseline (speedup 1.0000x reference)
import functools

import jax
import jax.numpy as jnp
from jax import lax
from jax.experimental import pallas as pl
from jax.experimental.pallas import tpu as pltpu
from jax.experimental.pallas import tpu_sc as plsc

_NC = 2
_NS = 16
_NW = _NC * _NS
_L = 16
_K = 128
_TC = 64


def _sc_mesh():
    return plsc.VectorSubcoreMesh(core_axis_name="c", subcore_axis_name="s",
                                  num_cores=_NC, num_subcores=_NS)


def _fill_zeros(ref, rows, width):
    zv = jnp.zeros((_L,), jnp.float32)

    def row(r, carry):
        for k in range(width // _L):
            ref[r, pl.ds(k * _L, _L)] = zv
        return carry

    lax.fori_loop(0, rows, row, 0)


@functools.partial(jax.jit, static_argnums=(1, 2))
def _bincount(dst2, NP, EP):
    CW = EP // _K // _NW
    RT = NP // _NS

    def body(dst_hbm, out_hbm, dst_v, ones_v, zbuf, acc):
        cid = lax.axis_index("c")
        sid = lax.axis_index("s")
        wid = sid * _NC + cid

        ov = jnp.ones((_L,), jnp.float32)

        def orow(r, carry):
            ones_v[r, pl.ds(0, _L)] = ov
            return carry

        lax.fori_loop(0, _K, orow, 0)
        _fill_zeros(zbuf, RT, _L)

        pltpu.sync_copy(dst_hbm.at[pl.ds(wid * CW, CW)], dst_v)
        pltpu.sync_copy(zbuf, acc.at[pl.ds(sid * RT, RT)])
        plsc.subcore_barrier()

        def step(j, carry):
            pltpu.sync_copy(ones_v, acc.at[dst_v.at[j]], add=True)
            return carry

        lax.fori_loop(0, CW, step, 0)
        plsc.subcore_barrier()
        pltpu.sync_copy(acc.at[pl.ds(sid * RT, RT)],
                        out_hbm.at[cid, pl.ds(sid * RT, RT)])

    f = pl.kernel(
        body,
        out_type=jax.ShapeDtypeStruct((_NC, NP, _L), jnp.float32),
        mesh=_sc_mesh(),
        compiler_params=pltpu.CompilerParams(use_tc_tiling_on_sc=False),
        scratch_types=[
            pltpu.VMEM((CW, _K), jnp.int32),
            pltpu.VMEM((_K, _L), jnp.float32),
            pltpu.VMEM((RT, _L), jnp.float32),
            pltpu.VMEM_SHARED((NP, _L), jnp.float32),
        ],
    )
    return f(dst2)


@functools.partial(jax.jit, static_argnums=(3, 4, 5, 6))
def _segsum(tables, src2, dst2, n_tables, NP, EP, SPLIT8):
    CH16 = EP // _K // _NS
    CW0 = CH16 * SPLIT8 // 8
    CW1 = CH16 - CW0
    C0T = _NS * CW0
    CW = max(CW0, CW1)
    RT = NP // _NS
    ZR = next(d for d in (32, 16, 8, RT) if RT % d == 0)

    NB = 4
    LD = 2

    def body(*refs):
        tabs = refs[:n_tables]
        src_hbm, dst_hbm, out_hbm = refs[n_tables:n_tables + 3]
        src_v, dst_v, rows_v, zbuf, acc, gsem = refs[n_tables + 3:]

        cid = lax.axis_index("c")
        sid = lax.axis_index("s")

        def gather(tab, c, b):
            return pltpu.make_async_copy(tab.at[src_v.at[c]], rows_v.at[b],
                                         gsem.at[b])

        _fill_zeros(zbuf, ZR, _TC)
        cw = jnp.where(cid == 0, CW0, CW1)

        @pl.when(cid == 0)
        def _():
            pltpu.sync_copy(src_hbm.at[pl.ds(sid * CW0, CW0)],
                            src_v.at[pl.ds(0, CW0)])
            pltpu.sync_copy(dst_hbm.at[pl.ds(sid * CW0, CW0)],
                            dst_v.at[pl.ds(0, CW0)])

        @pl.when(cid == 1)
        def _():
            pltpu.sync_copy(src_hbm.at[pl.ds(C0T + sid * CW1, CW1)],
                            src_v.at[pl.ds(0, CW1)])
            pltpu.sync_copy(dst_hbm.at[pl.ds(C0T + sid * CW1, CW1)],
                            dst_v.at[pl.ds(0, CW1)])

        for t in range(n_tables):
            tab = tabs[t]
            for c in range(LD):
                @pl.when(c < cw)
                def _(c=c):
                    gather(tab, c, c).start()
            for z in range(RT // ZR):
                pltpu.sync_copy(zbuf, acc.at[pl.ds(sid * RT + z * ZR, ZR)])
            plsc.subcore_barrier()

            @pl.loop(0, cw, step=NB)
            def _chunks(j):
                for i in range(NB):
                    jb = j + i
                    c = jb + LD
                    bg = (i + LD) % NB

                    @pl.when(c < cw)
                    def _():
                        gather(tab, c, bg).start()

                    gather(tab, jb, i).wait()
                    pltpu.sync_copy(rows_v.at[i], acc.at[dst_v.at[jb]],
                                    add=True)

            plsc.subcore_barrier()
            pltpu.sync_copy(acc.at[pl.ds(sid * RT, RT)],
                            out_hbm.at[cid, t, pl.ds(sid * RT, RT)])
            if t + 1 < n_tables:
                plsc.subcore_barrier()

    f = pl.kernel(
        body,
        out_type=jax.ShapeDtypeStruct((_NC, n_tables, NP, _TC), jnp.float32),
        mesh=_sc_mesh(),
        compiler_params=pltpu.CompilerParams(use_tc_tiling_on_sc=False),
        scratch_types=[
            pltpu.VMEM((CW, _K), jnp.int32),
            pltpu.VMEM((CW, _K), jnp.int32),
            pltpu.VMEM((NB, _K, _TC), jnp.float32),
            pltpu.VMEM((ZR, _TC), jnp.float32),
            pltpu.VMEM_SHARED((NP, _TC), jnp.float32),
            pltpu.SemaphoreType.DMA((NB,)),
        ],
    )
    return f(*tables, src2, dst2)


def _row_block(N):
    for cand in (400, 500, 250, 200, 128, 100, 80, 50, 40, 25, 20, 16, 10, 8, 5, 4, 2, 1):
        if N % cand == 0:
            return cand
    return 1


def _disg0_call(degp, x, N, BR):
    F = x.shape[1]
    NT = F // _TC

    def body(degp_ref, x_ref, dis_ref, *outs):
        p = degp_ref[...]
        deg = 1.0 + p[0, :, 0:1] + p[1, :, 0:1]
        dis = lax.rsqrt(deg)
        dis_ref[...] = dis
        g0 = x_ref[...] * dis
        for c in range(NT):
            outs[c][...] = g0[:, c * _TC:(c + 1) * _TC]

    return pl.pallas_call(
        body,
        grid=(N // BR,),
        in_specs=[
            pl.BlockSpec((_NC, BR, _L), lambda i: (0, i, 0)),
            pl.BlockSpec((BR, F), lambda i: (i, 0)),
        ],
        out_specs=[pl.BlockSpec((BR, 1), lambda i: (i, 0))]
        + [pl.BlockSpec((BR, _TC), lambda i: (i, 0))] * NT,
        out_shape=[jax.ShapeDtypeStruct((N, 1), jnp.float32)]
        + [jax.ShapeDtypeStruct((N, _TC), jnp.float32)] * NT,
    )(degp, x)


def _conv1_call(s0p, g0s, dis, W1, b1, W2, N, BR):
    NT0 = len(g0s)
    F = NT0 * _TC
    H1 = W1.shape[1]
    H2 = W2.shape[1]
    NT1 = H2 // _TC

    def body(s0p_ref, *refs):
        g0_refs = refs[:NT0]
        dis_ref, W1_ref, b1_ref, W2_ref = refs[NT0:NT0 + 4]
        outs = refs[NT0 + 4:]
        dis = dis_ref[...]
        s = jnp.concatenate(
            [s0p_ref[0, c] + s0p_ref[1, c] + g0_refs[c][...]
             for c in range(NT0)], axis=1)
        a = dis * s
        h1 = jnp.maximum(
            jnp.dot(a, W1_ref[...], preferred_element_type=jnp.float32)
            + b1_ref[...], 0.0)
        g1 = dis * jnp.dot(h1, W2_ref[...], preferred_element_type=jnp.float32)
        for c in range(NT1):
            outs[c][...] = g1[:, c * _TC:(c + 1) * _TC]

    return pl.pallas_call(
        body,
        grid=(N // BR,),
        in_specs=[pl.BlockSpec((_NC, NT0, BR, _TC), lambda i: (0, 0, i, 0))]
        + [pl.BlockSpec((BR, _TC), lambda i: (i, 0))] * NT0
        + [
            pl.BlockSpec((BR, 1), lambda i: (i, 0)),
            pl.BlockSpec((F, H1), lambda i: (0, 0)),
            pl.BlockSpec((1, H1), lambda i: (0, 0)),
            pl.BlockSpec((H1, H2), lambda i: (0, 0)),
        ],
        out_specs=[pl.BlockSpec((BR, _TC), lambda i: (i, 0))] * NT1,
        out_shape=[jax.ShapeDtypeStruct((N, _TC), jnp.float32)] * NT1,
    )(s0p, *g0s, dis, W1, b1, W2)


def _conv2_call(s1p, g1s, dis, b2, W3, N, BR):
    NT1 = len(g1s)
    H2 = NT1 * _TC
    H3 = W3.shape[1]
    NT2 = H3 // _TC

    def body(s1p_ref, *refs):
        g1_refs = refs[:NT1]
        dis_ref, b2_ref, W3_ref = refs[NT1:NT1 + 3]
        outs = refs[NT1 + 3:]
        dis = dis_ref[...]
        s = jnp.concatenate(
            [s1p_ref[0, c] + s1p_ref[1, c] + g1_refs[c][...]
             for c in range(NT1)], axis=1)
        h2 = jnp.maximum(dis * s + b2_ref[...], 0.0)
        g2 = dis * jnp.dot(h2, W3_ref[...], preferred_element_type=jnp.float32)
        for c in range(NT2):
            outs[c][...] = g2[:, c * _TC:(c + 1) * _TC]

    return pl.pallas_call(
        body,
        grid=(N // BR,),
        in_specs=[pl.BlockSpec((_NC, NT1, BR, _TC), lambda i: (0, 0, i, 0))]
        + [pl.BlockSpec((BR, _TC), lambda i: (i, 0))] * NT1
        + [
            pl.BlockSpec((BR, 1), lambda i: (i, 0)),
            pl.BlockSpec((1, H2), lambda i: (0, 0)),
            pl.BlockSpec((H2, H3), lambda i: (0, 0)),
        ],
        out_specs=[pl.BlockSpec((BR, _TC), lambda i: (i, 0))] * NT2,
        out_shape=[jax.ShapeDtypeStruct((N, _TC), jnp.float32)] * NT2,
    )(s1p, *g1s, dis, b2, W3)


def _conv3_call(s2p, g2s, dis, b3, Wfc, bfc, N, BR):
    NT2 = len(g2s)
    H3 = NT2 * _TC
    C = Wfc.shape[1]

    def body(s2p_ref, *refs):
        g2_refs = refs[:NT2]
        dis_ref, b3_ref, Wfc_ref, bfc_ref, out_ref = refs[NT2:]
        dis = dis_ref[...]
        s = jnp.concatenate(
            [s2p_ref[0, c] + s2p_ref[1, c] + g2_refs[c][...]
             for c in range(NT2)], axis=1)
        h3 = jnp.maximum(dis * s + b3_ref[...], 0.0)
        out_ref[...] = (
            jnp.dot(h3, Wfc_ref[...], preferred_element_type=jnp.float32)
            + bfc_ref[...])

    return pl.pallas_call(
        body,
        grid=(N // BR,),
        in_specs=[pl.BlockSpec((_NC, NT2, BR, _TC), lambda i: (0, 0, i, 0))]
        + [pl.BlockSpec((BR, _TC), lambda i: (i, 0))] * NT2
        + [
            pl.BlockSpec((BR, 1), lambda i: (i, 0)),
            pl.BlockSpec((1, H3), lambda i: (0, 0)),
            pl.BlockSpec((H3, C), lambda i: (0, 0)),
            pl.BlockSpec((1, C), lambda i: (0, 0)),
        ],
        out_specs=pl.BlockSpec((BR, C), lambda i: (i, 0)),
        out_shape=jax.ShapeDtypeStruct((N, C), jnp.float32),
    )(s2p, *g2s, dis, b3, Wfc, bfc)


def kernel(x, edge_index, W1, b1, W2, b2, W3, b3, Wfc, bfc):
    N, F = x.shape
    E = edge_index.shape[1]
    H2 = W2.shape[1]

    grain = _NW * _K * 8
    EP = ((E + grain - 1) // grain) * grain
    NP = ((N + 1 + 2047) // 2048) * 2048
    BR = _row_block(N)

    src = edge_index[0]
    dst = edge_index[1]
    if EP > E:
        pad = EP - E
        src = jnp.concatenate([src, jnp.zeros((pad,), jnp.int32)])
        dst = jnp.concatenate([dst, jnp.full((pad,), N, jnp.int32)])
    src2 = src.reshape(EP // _K, _K)
    dst2 = dst.reshape(EP // _K, _K)

    degp = _bincount(dst2, NP, EP)
    dis, *g0s = _disg0_call(degp, x, N, BR)

    s0p = _segsum(tuple(g0s), src2, dst2, len(g0s), NP, EP, 6)
    g1s = _conv1_call(s0p, g0s, dis, W1, b1.reshape(1, -1), W2, N, BR)

    s1p = _segsum(tuple(g1s), src2, dst2, len(g1s), NP, EP, 6)
    g2s = _conv2_call(s1p, g1s, dis, b2.reshape(1, -1), W3, N, BR)

    s2p = _segsum(tuple(g2s), src2, dst2, len(g2s), NP, EP, 6)
    out = _conv3_call(s2p, g2s, dis, b3.reshape(1, -1), Wfc,
                      bfc.reshape(1, -1), N, BR)
    return out
